# baseline (speedup 1.0000x reference)
import functools

import jax
import jax.numpy as jnp
from jax import lax
from jax.experimental import pallas as pl
from jax.experimental.pallas import tpu as pltpu
from jax.experimental.pallas import tpu_sc as plsc

NW = 32
K = 128
DW = 16


def _mesh():
    return plsc.VectorSubcoreMesh(core_axis_name="c", subcore_axis_name="s")


def _row_chunks(rows):
    sizes = []
    left = rows
    while left > 0:
        sizes.append(min(K, left))
        left -= sizes[-1]
    return sizes


def _make_deg_kernel(ch, np_):
    rpt = np_ // 16
    sizes = _row_chunks(rpt)

    @functools.partial(
        pl.kernel,
        mesh=_mesh(),
        out_type=jax.ShapeDtypeStruct((2, np_, DW), jnp.float32),
        scratch_types=[
            pltpu.VMEM((ch, K), jnp.int32),
            pltpu.VMEM((ch, K), jnp.float32),
            pltpu.VMEM((K, DW), jnp.float32),
            pltpu.VMEM_SHARED((np_, DW), jnp.float32),
        ],
    )
    def deg_kernel(dst_hbm, w_hbm, out_hbm, dst_v, w_v, wrow_v, deg_sh):
        cid = lax.axis_index("c")
        sid = lax.axis_index("s")
        wid = sid * 2 + cid
        pltpu.sync_copy(dst_hbm.at[wid], dst_v)
        pltpu.sync_copy(w_hbm.at[wid], w_v)
        zero = jnp.zeros((16,), jnp.float32)

        def zb(i, carry):
            wrow_v[i, pl.ds(0, DW)] = zero[:DW]
            return carry

        lax.fori_loop(0, K, zb, 0)
        base = sid * rpt
        off = 0
        for sz in sizes:
            pltpu.sync_copy(wrow_v.at[pl.ds(0, sz)],
                            deg_sh.at[pl.ds(base + off, sz)])
            off += sz
        plsc.subcore_barrier()

        def chunk(c, carry):
            def cb(v, carry2):
                wv = w_v[c, pl.ds(v * 16, 16)]
                for j in range(16):
                    s = wv[j]
                    wrow_v[v * 16 + j, pl.ds(0, DW)] = jnp.full(
                        (DW,), s, jnp.float32)
                return carry2

            lax.fori_loop(0, K // 16, cb, 0)
            pltpu.sync_copy(wrow_v, deg_sh.at[dst_v.at[c]], add=True)
            return carry

        lax.fori_loop(0, ch, chunk, 0)
        plsc.subcore_barrier()
        off = 0
        for sz in sizes:
            pltpu.sync_copy(deg_sh.at[pl.ds(base + off, sz)],
                            out_hbm.at[cid, pl.ds(base + off, sz)])
            off += sz

    return deg_kernel


def _make_agg_kernel(ch, np_):
    rpt = np_ // 16
    sizes = _row_chunks(rpt)

    @functools.partial(
        pl.kernel,
        mesh=_mesh(),
        out_type=jax.ShapeDtypeStruct((2, np_, 128), jnp.float32),
        scratch_types=[
            pltpu.VMEM((ch + 2, K), jnp.int32),
            pltpu.VMEM((ch + 2, K), jnp.int32),
            pltpu.VMEM((ch + 2, K), jnp.float32),
            pltpu.VMEM((K, 128), jnp.float32),
            pltpu.VMEM_SHARED((np_, 128), jnp.float32),
            pltpu.SemaphoreType.DMA,
        ],
    )
    def agg_kernel(g_hbm, src_hbm, dst_hbm, w_hbm, out_hbm,
                   src_v, dst_v, w_v, rows_v, s_sh, sem):
        cid = lax.axis_index("c")
        sid = lax.axis_index("s")
        wid = sid * 2 + cid
        pltpu.sync_copy(src_hbm.at[wid], src_v)
        pltpu.sync_copy(dst_hbm.at[wid], dst_v)
        pltpu.sync_copy(w_hbm.at[wid], w_v)
        zero = jnp.zeros((16,), jnp.float32)

        def zb(i, carry):
            rows_v[i // 8, pl.ds((i % 8) * 16, 16)] = zero
            return carry

        lax.fori_loop(0, K * 8, zb, 0)
        base = sid * rpt
        off = 0
        for sz in sizes:
            pltpu.sync_copy(rows_v.at[pl.ds(0, sz)],
                            s_sh.at[pl.ds(base + off, sz)])
            off += sz
        plsc.subcore_barrier()

        def chunk(c, carry):
            pltpu.async_copy(g_hbm.at[src_v.at[c]], rows_v, sem).wait()

            def sb(v, carry2):
                wv = w_v[c, pl.ds(v * 16, 16)]
                for j in range(16):
                    s = wv[j]
                    e = v * 16 + j
                    for f in range(8):
                        sl = pl.ds(f * 16, 16)
                        rows_v[e, sl] = rows_v[e, sl] * s
                return carry2

            lax.fori_loop(0, K // 16, sb, 0)
            pltpu.sync_copy(rows_v, s_sh.at[dst_v.at[c]], add=True)
            return carry

        lax.fori_loop(0, ch, chunk, 0)
        plsc.subcore_barrier()
        off = 0
        for sz in sizes:
            pltpu.sync_copy(s_sh.at[pl.ds(base + off, sz)],
                            out_hbm.at[cid, pl.ds(base + off, sz)])
            off += sz

    return agg_kernel


def _dinv_col(dp_ref):
    deg = dp_ref[0, :, 0:1] + dp_ref[1, :, 0:1] + 1.0
    return lax.rsqrt(deg)


def _tc_prep(xpad, W1, degp):
    np_ = xpad.shape[0]

    def body(x_ref, w_ref, dp_ref, g_ref):
        dinv = _dinv_col(dp_ref)
        h = jnp.dot(x_ref[...], w_ref[...], preferred_element_type=jnp.float32)
        g_ref[...] = h * dinv

    return pl.pallas_call(
        body,
        out_shape=jax.ShapeDtypeStruct((np_, 128), jnp.float32),
    )(xpad, W1, degp)


def _tc_mid(degp, sparts, g1, b1r, W2):
    np_ = g1.shape[0]

    def body(dp_ref, sp_ref, g_ref, b_ref, w_ref, o_ref):
        dinv = _dinv_col(dp_ref)
        x = jnp.maximum(
            dinv * (sp_ref[0] + sp_ref[1] + g_ref[...]) + b_ref[...], 0.0)
        o_ref[...] = jnp.dot(
            x, w_ref[...], preferred_element_type=jnp.float32) * dinv

    return pl.pallas_call(
        body,
        out_shape=jax.ShapeDtypeStruct((np_, 128), jnp.float32),
    )(degp, sparts, g1, b1r, W2)


def _tc_head(degp, sparts, g2, n, b2r, fc1_W, fc1br, fc2_W, fc2br):
    def body(dp_ref, sp_ref, g_ref, b_ref, w1_ref, b1_ref, w2_ref, b2_ref,
             out_ref):
        dinv = _dinv_col(dp_ref)
        x = jnp.maximum(
            dinv * (sp_ref[0] + sp_ref[1] + g_ref[...]) + b_ref[...], 0.0)
        x = x[:n, :]
        mean_x = jnp.sum(x, axis=0, keepdims=True) * (1.0 / n)
        max_x = jnp.max(x, axis=0, keepdims=True)
        h = jnp.concatenate([mean_x, max_x], axis=1)
        h = jnp.maximum(
            jnp.dot(h, w1_ref[...], preferred_element_type=jnp.float32)
            + b1_ref[...], 0.0)
        out_ref[...] = (
            jnp.dot(h, w2_ref[...], preferred_element_type=jnp.float32)
            + b2_ref[...])

    return pl.pallas_call(
        body,
        out_shape=jax.ShapeDtypeStruct((1, fc2_W.shape[1]), jnp.float32),
    )(degp, sparts, g2, b2r, fc1_W, fc1br, fc2_W, fc2br)


def kernel(emb_x, edge_index, edge_weight, W1, b1, W2, b2,
           fc1_W, fc1_b, fc2_W, fc2_b):
    n, d = emb_x.shape
    e = edge_weight.shape[0]

    ch = -(-e // (NW * K))
    ch += ch % 2
    cht = ch + 2
    pad = NW * ch * K - e

    def _chunked(a):
        a = jnp.pad(a, (0, pad)).reshape(NW, ch, K)
        return jnp.pad(a, ((0, 0), (0, 2), (0, 0)))

    src2 = _chunked(edge_index[0])
    dst2 = _chunked(edge_index[1])
    w2 = _chunked(edge_weight)

    np_ = -(-n // 128) * 128
    xpad = jnp.pad(emb_x, ((0, np_ - n), (0, 0)))

    degp = _make_deg_kernel(cht, np_)(dst2, w2)
    g1 = _tc_prep(xpad, W1, degp)

    agg = _make_agg_kernel(ch, np_)
    s1 = agg(g1, src2, dst2, w2)
    g2 = _tc_mid(degp, s1, g1, b1.reshape(1, -1), W2)
    s2 = agg(g2, src2, dst2, w2)

    return _tc_head(degp, s2, g2, n, b2.reshape(1, -1), fc1_W,
                    fc1_b.reshape(1, -1), fc2_W, fc2_b.reshape(1, -1))


# speedup vs baseline: 1.3542x; 1.3542x over previous
import functools

import jax
import jax.numpy as jnp
from jax import lax
from jax.experimental import pallas as pl
from jax.experimental.pallas import tpu as pltpu
from jax.experimental.pallas import tpu_sc as plsc

NW = 32
K = 128
DW = 16


def _mesh():
    return plsc.VectorSubcoreMesh(core_axis_name="c", subcore_axis_name="s")


def _row_chunks(rows):
    sizes = []
    left = rows
    while left > 0:
        sizes.append(min(K, left))
        left -= sizes[-1]
    return sizes


def _make_deg_kernel(ch, np_):
    rpt = np_ // 16
    sizes = _row_chunks(rpt)

    @functools.partial(
        pl.kernel,
        mesh=_mesh(),
        out_type=jax.ShapeDtypeStruct((2, np_, DW), jnp.float32),
        scratch_types=[
            pltpu.VMEM((ch, K), jnp.int32),
            pltpu.VMEM((ch, K), jnp.float32),
            pltpu.VMEM((K, DW), jnp.float32),
            pltpu.VMEM_SHARED((np_, DW), jnp.float32),
        ],
    )
    def deg_kernel(dst_hbm, w_hbm, out_hbm, dst_v, w_v, wrow_v, deg_sh):
        cid = lax.axis_index("c")
        sid = lax.axis_index("s")
        wid = sid * 2 + cid
        pltpu.sync_copy(dst_hbm.at[wid], dst_v)
        pltpu.sync_copy(w_hbm.at[wid], w_v)
        zero = jnp.zeros((16,), jnp.float32)

        def zb(i, carry):
            wrow_v[i, pl.ds(0, DW)] = zero[:DW]
            return carry

        lax.fori_loop(0, K, zb, 0)
        base = sid * rpt
        off = 0
        for sz in sizes:
            pltpu.sync_copy(wrow_v.at[pl.ds(0, sz)],
                            deg_sh.at[pl.ds(base + off, sz)])
            off += sz
        plsc.subcore_barrier()

        def chunk(c, carry):
            def cb(v, carry2):
                wv = w_v[c, pl.ds(v * 16, 16)]
                for j in range(16):
                    s = wv[j]
                    wrow_v[v * 16 + j, pl.ds(0, DW)] = jnp.full(
                        (DW,), s, jnp.float32)
                return carry2

            lax.fori_loop(0, K // 16, cb, 0)
            pltpu.sync_copy(wrow_v, deg_sh.at[dst_v.at[c]], add=True)
            return carry

        lax.fori_loop(0, ch, chunk, 0)
        plsc.subcore_barrier()
        off = 0
        for sz in sizes:
            pltpu.sync_copy(deg_sh.at[pl.ds(base + off, sz)],
                            out_hbm.at[cid, pl.ds(base + off, sz)])
            off += sz

    return deg_kernel


def _make_agg_kernel(ch, np_):
    rpt = np_ // 16
    sizes = _row_chunks(rpt)

    @functools.partial(
        pl.kernel,
        mesh=_mesh(),
        out_type=jax.ShapeDtypeStruct((2, np_, 128), jnp.float32),
        scratch_types=[
            pltpu.VMEM((ch, K), jnp.int32),
            pltpu.VMEM((ch, K), jnp.int32),
            pltpu.VMEM((ch, K), jnp.float32),
            pltpu.VMEM((K, 128), jnp.float32),
            pltpu.VMEM_SHARED((np_, 128), jnp.float32),
            pltpu.SemaphoreType.DMA,
        ],
    )
    def agg_kernel(g_hbm, src_hbm, dst_hbm, w_hbm, out_hbm,
                   src_v, dst_v, w_v, rows_v, s_sh, sem):
        cid = lax.axis_index("c")
        sid = lax.axis_index("s")
        wid = sid * 2 + cid
        pltpu.sync_copy(src_hbm.at[wid], src_v)
        pltpu.sync_copy(dst_hbm.at[wid], dst_v)
        pltpu.sync_copy(w_hbm.at[wid], w_v)
        zero = jnp.zeros((16,), jnp.float32)

        def zb(i, carry):
            rows_v[i // 8, pl.ds((i % 8) * 16, 16)] = zero
            return carry

        lax.fori_loop(0, K * 8, zb, 0)
        base = sid * rpt
        off = 0
        for sz in sizes:
            pltpu.sync_copy(rows_v.at[pl.ds(0, sz)],
                            s_sh.at[pl.ds(base + off, sz)])
            off += sz
        plsc.subcore_barrier()

        def chunk(c, carry):
            pltpu.async_copy(g_hbm.at[src_v.at[c]], rows_v, sem).wait()

            def sb(v, carry2):
                wv = w_v[c, pl.ds(v * 16, 16)]
                for j in range(16):
                    s = wv[j]
                    e = v * 16 + j
                    for f in range(8):
                        sl = pl.ds(f * 16, 16)
                        rows_v[e, sl] = rows_v[e, sl] * s
                return carry2

            lax.fori_loop(0, K // 16, sb, 0)
            pltpu.sync_copy(rows_v, s_sh.at[dst_v.at[c]], add=True)
            return carry

        lax.fori_loop(0, ch, chunk, 0)
        plsc.subcore_barrier()
        off = 0
        for sz in sizes:
            pltpu.sync_copy(s_sh.at[pl.ds(base + off, sz)],
                            out_hbm.at[cid, pl.ds(base + off, sz)])
            off += sz

    return agg_kernel


def _dinv_col(dp_ref):
    deg = dp_ref[0, :, 0:1] + dp_ref[1, :, 0:1] + 1.0
    return lax.rsqrt(deg)


def _tc_prep(xpad, W1, degp):
    np_ = xpad.shape[0]

    def body(x_ref, w_ref, dp_ref, g_ref):
        dinv = _dinv_col(dp_ref)
        h = jnp.dot(x_ref[...], w_ref[...], preferred_element_type=jnp.float32)
        g_ref[...] = h * dinv

    return pl.pallas_call(
        body,
        out_shape=jax.ShapeDtypeStruct((np_, 128), jnp.float32),
    )(xpad, W1, degp)


def _tc_mid(degp, sparts, g1, b1r, W2):
    np_ = g1.shape[0]

    def body(dp_ref, sp_ref, g_ref, b_ref, w_ref, o_ref):
        dinv = _dinv_col(dp_ref)
        x = jnp.maximum(
            dinv * (sp_ref[0] + sp_ref[1] + g_ref[...]) + b_ref[...], 0.0)
        o_ref[...] = jnp.dot(
            x, w_ref[...], preferred_element_type=jnp.float32) * dinv

    return pl.pallas_call(
        body,
        out_shape=jax.ShapeDtypeStruct((np_, 128), jnp.float32),
    )(degp, sparts, g1, b1r, W2)


def _tc_head(degp, sparts, g2, n, b2r, fc1_W, fc1br, fc2_W, fc2br):
    def body(dp_ref, sp_ref, g_ref, b_ref, w1_ref, b1_ref, w2_ref, b2_ref,
             out_ref):
        dinv = _dinv_col(dp_ref)
        x = jnp.maximum(
            dinv * (sp_ref[0] + sp_ref[1] + g_ref[...]) + b_ref[...], 0.0)
        x = x[:n, :]
        mean_x = jnp.sum(x, axis=0, keepdims=True) * (1.0 / n)
        max_x = jnp.max(x, axis=0, keepdims=True)
        h = jnp.concatenate([mean_x, max_x], axis=1)
        h = jnp.maximum(
            jnp.dot(h, w1_ref[...], preferred_element_type=jnp.float32)
            + b1_ref[...], 0.0)
        out_ref[...] = (
            jnp.dot(h, w2_ref[...], preferred_element_type=jnp.float32)
            + b2_ref[...])

    return pl.pallas_call(
        body,
        out_shape=jax.ShapeDtypeStruct((1, fc2_W.shape[1]), jnp.float32),
    )(degp, sparts, g2, b2r, fc1_W, fc1br, fc2_W, fc2br)


def kernel(emb_x, edge_index, edge_weight, W1, b1, W2, b2,
           fc1_W, fc1_b, fc2_W, fc2_b):
    n, d = emb_x.shape
    e = edge_weight.shape[0]

    ch = -(-e // (NW * K))
    pad = NW * ch * K - e
    src2 = jnp.pad(edge_index[0], (0, pad)).reshape(NW, ch, K)
    dst2 = jnp.pad(edge_index[1], (0, pad)).reshape(NW, ch, K)
    w2 = jnp.pad(edge_weight, (0, pad)).reshape(NW, ch, K)

    np_ = -(-n // 128) * 128
    xpad = jnp.pad(emb_x, ((0, np_ - n), (0, 0)))

    degp = _make_deg_kernel(ch, np_)(dst2, w2)
    g1 = _tc_prep(xpad, W1, degp)

    agg = _make_agg_kernel(ch, np_)
    s1 = agg(g1, src2, dst2, w2)
    g2 = _tc_mid(degp, s1, g1, b1.reshape(1, -1), W2)
    s2 = agg(g2, src2, dst2, w2)

    return _tc_head(degp, s2, g2, n, b2.reshape(1, -1), fc1_W,
                    fc1_b.reshape(1, -1), fc2_W, fc2_b.reshape(1, -1))


# speedup vs baseline: 2.1637x; 1.5978x over previous
import functools

import jax
import jax.numpy as jnp
from jax import lax
from jax.experimental import pallas as pl
from jax.experimental.pallas import tpu as pltpu
from jax.experimental.pallas import tpu_sc as plsc

NW = 32
K = 128
DW = 16


def _mesh():
    return plsc.VectorSubcoreMesh(core_axis_name="c", subcore_axis_name="s")


def _row_chunks(rows):
    sizes = []
    left = rows
    while left > 0:
        sizes.append(min(K, left))
        left -= sizes[-1]
    return sizes


def _make_deg_kernel(ch, np_):
    rpt = np_ // 16
    sizes = _row_chunks(rpt)

    @functools.partial(
        pl.kernel,
        mesh=_mesh(),
        out_type=jax.ShapeDtypeStruct((2, np_, DW), jnp.float32),
        scratch_types=[
            pltpu.VMEM((ch, K), jnp.int32),
            pltpu.VMEM((ch, K), jnp.float32),
            pltpu.VMEM((K, DW), jnp.float32),
            pltpu.VMEM_SHARED((np_, DW), jnp.float32),
        ],
    )
    def deg_kernel(dst_hbm, w_hbm, out_hbm, dst_v, w_v, wrow_v, deg_sh):
        cid = lax.axis_index("c")
        sid = lax.axis_index("s")
        wid = sid * 2 + cid
        pltpu.sync_copy(dst_hbm.at[wid], dst_v)
        pltpu.sync_copy(w_hbm.at[wid], w_v)
        zero = jnp.zeros((16,), jnp.float32)

        def zb(i, carry):
            wrow_v[i, pl.ds(0, DW)] = zero[:DW]
            return carry

        lax.fori_loop(0, K, zb, 0)
        base = sid * rpt
        off = 0
        for sz in sizes:
            pltpu.sync_copy(wrow_v.at[pl.ds(0, sz)],
                            deg_sh.at[pl.ds(base + off, sz)])
            off += sz
        plsc.subcore_barrier()

        def chunk(c, carry):
            def cb(v, carry2):
                wv = w_v[c, pl.ds(v * 16, 16)]
                for j in range(16):
                    s = wv[j]
                    wrow_v[v * 16 + j, pl.ds(0, DW)] = jnp.full(
                        (DW,), s, jnp.float32)
                return carry2

            lax.fori_loop(0, K // 16, cb, 0)
            pltpu.sync_copy(wrow_v, deg_sh.at[dst_v.at[c]], add=True)
            return carry

        lax.fori_loop(0, ch, chunk, 0)
        plsc.subcore_barrier()
        off = 0
        for sz in sizes:
            pltpu.sync_copy(deg_sh.at[pl.ds(base + off, sz)],
                            out_hbm.at[cid, pl.ds(base + off, sz)])
            off += sz

    return deg_kernel


def _make_agg_kernel(ch, np_):
    rpt = np_ // 16
    sizes = _row_chunks(rpt)

    @functools.partial(
        pl.kernel,
        mesh=_mesh(),
        out_type=jax.ShapeDtypeStruct((2, np_, 128), jnp.float32),
        scratch_types=[
            pltpu.VMEM((ch, K), jnp.int32),
            pltpu.VMEM((ch, K), jnp.int32),
            pltpu.VMEM((ch, K), jnp.float32),
            pltpu.VMEM((K, 128), jnp.float32),
            pltpu.VMEM_SHARED((np_, 128), jnp.float32),
            pltpu.SemaphoreType.DMA,
        ],
    )
    def agg_kernel(g_hbm, src_hbm, dst_hbm, w_hbm, out_hbm,
                   src_v, dst_v, w_v, rows_v, s_sh, sem):
        cid = lax.axis_index("c")
        sid = lax.axis_index("s")
        wid = sid * 2 + cid
        pltpu.sync_copy(src_hbm.at[wid], src_v)
        pltpu.sync_copy(dst_hbm.at[wid], dst_v)
        pltpu.sync_copy(w_hbm.at[wid], w_v)
        zero = jnp.zeros((16,), jnp.float32)

        def zb(i, carry):
            rows_v[i // 8, pl.ds((i % 8) * 16, 16)] = zero
            return carry

        lax.fori_loop(0, K * 8, zb, 0)
        base = sid * rpt
        off = 0
        for sz in sizes:
            pltpu.sync_copy(rows_v.at[pl.ds(0, sz)],
                            s_sh.at[pl.ds(base + off, sz)])
            off += sz
        plsc.subcore_barrier()

        def chunk(c, carry):
            pltpu.async_copy(g_hbm.at[src_v.at[c]], rows_v, sem).wait()

            def sb(v, carry2):
                wv = w_v[c, pl.ds(v * 16, 16)]
                for j in range(16):
                    s = wv[j]
                    e = v * 16 + j
                    for f in range(8):
                        sl = pl.ds(f * 16, 16)
                        rows_v[e, sl] = rows_v[e, sl] * s
                return carry2

            lax.fori_loop(0, K // 16, sb, 0)
            pltpu.sync_copy(rows_v, s_sh.at[dst_v.at[c]], add=True)
            return carry

        lax.fori_loop(0, ch, chunk, 0)
        plsc.subcore_barrier()
        off = 0
        for sz in sizes:
            pltpu.sync_copy(s_sh.at[pl.ds(base + off, sz)],
                            out_hbm.at[cid, pl.ds(base + off, sz)])
            off += sz

    return agg_kernel


def _dinv_col(dp_ref):
    deg = dp_ref[0, :, 0:1] + dp_ref[1, :, 0:1] + 1.0
    return lax.rsqrt(deg)


def _tc_prep(xpad, W1, degp):
    np_ = xpad.shape[0]

    def body(x_ref, w_ref, dp_ref, g_ref):
        dinv = _dinv_col(dp_ref)
        h = jnp.dot(x_ref[...], w_ref[...], preferred_element_type=jnp.float32)
        g_ref[...] = h * dinv

    return pl.pallas_call(
        body,
        out_shape=jax.ShapeDtypeStruct((np_, 128), jnp.float32),
    )(xpad, W1, degp)


def _tc_mid(degp, sparts, g1, b1r, W2):
    np_ = g1.shape[0]

    def body(dp_ref, sp_ref, g_ref, b_ref, w_ref, o_ref):
        dinv = _dinv_col(dp_ref)
        x = jnp.maximum(
            dinv * (sp_ref[0] + sp_ref[1] + g_ref[...]) + b_ref[...], 0.0)
        o_ref[...] = jnp.dot(
            x, w_ref[...], preferred_element_type=jnp.float32) * dinv

    return pl.pallas_call(
        body,
        out_shape=jax.ShapeDtypeStruct((np_, 128), jnp.float32),
    )(degp, sparts, g1, b1r, W2)


def _tc_head(degp, sparts, g2, n, b2r, fc1_W, fc1br, fc2_W, fc2br):
    def body(dp_ref, sp_ref, g_ref, b_ref, w1_ref, b1_ref, w2_ref, b2_ref,
             out_ref):
        dinv = _dinv_col(dp_ref)
        x = jnp.maximum(
            dinv * (sp_ref[0] + sp_ref[1] + g_ref[...]) + b_ref[...], 0.0)
        x = x[:n, :]
        mean_x = jnp.sum(x, axis=0, keepdims=True) * (1.0 / n)
        max_x = jnp.max(x, axis=0, keepdims=True)
        h = jnp.concatenate([mean_x, max_x], axis=1)
        h = jnp.maximum(
            jnp.dot(h, w1_ref[...], preferred_element_type=jnp.float32)
            + b1_ref[...], 0.0)
        out_ref[...] = (
            jnp.dot(h, w2_ref[...], preferred_element_type=jnp.float32)
            + b2_ref[...])

    return pl.pallas_call(
        body,
        out_shape=jax.ShapeDtypeStruct((1, fc2_W.shape[1]), jnp.float32),
    )(degp, sparts, g2, b2r, fc1_W, fc1br, fc2_W, fc2br)


def kernel(emb_x, edge_index, edge_weight, W1, b1, W2, b2,
           fc1_W, fc1_b, fc2_W, fc2_b):
    n, d = emb_x.shape
    e = edge_weight.shape[0]

    ch = -(-e // (NW * K))
    pad = NW * ch * K - e
    pad_idx = jnp.arange(pad, dtype=jnp.int32) % n
    src2 = jnp.concatenate([edge_index[0], pad_idx]).reshape(NW, ch, K)
    dst2 = jnp.concatenate([edge_index[1], pad_idx]).reshape(NW, ch, K)
    w2 = jnp.pad(edge_weight, (0, pad)).reshape(NW, ch, K)

    np_ = -(-n // 128) * 128
    xpad = jnp.pad(emb_x, ((0, np_ - n), (0, 0)))

    degp = _make_deg_kernel(ch, np_)(dst2, w2)
    g1 = _tc_prep(xpad, W1, degp)

    agg = _make_agg_kernel(ch, np_)
    s1 = agg(g1, src2, dst2, w2)
    g2 = _tc_mid(degp, s1, g1, b1.reshape(1, -1), W2)
    s2 = agg(g2, src2, dst2, w2)

    return _tc_head(degp, s2, g2, n, b2.reshape(1, -1), fc1_W,
                    fc1_b.reshape(1, -1), fc2_W, fc2_b.reshape(1, -1))
